# Initial kernel scaffold; baseline (speedup 1.0000x reference)
import jax
import jax.numpy as jnp
from jax.experimental import pallas as pl


def kernel(input, embed):
    raise NotImplementedError("write your pallas kernel here")


# speedup vs baseline: 1.1142x; 1.1142x over previous
import functools

import jax
import jax.numpy as jnp
from jax import lax
from jax.experimental import pallas as pl
from jax.experimental.pallas import tpu as pltpu
from jax.experimental.pallas import tpu_sc as plsc

_DIM = 32
_TN = 512
_TK = 2048


def _dist_argmin_body(x_ref, e_ref, idx_ref, loss_ref):
    i = pl.program_id(0)
    nsteps = pl.num_programs(0)
    x = x_ref[...]
    xx = jnp.sum(x ** 2, axis=1, keepdims=True)
    kk = e_ref.shape[1]
    nk = kk // _TK

    xb = x.astype(jnp.bfloat16)

    def body(k, carry):
        run_min, run_idx = carry
        e = e_ref[:, pl.ds(k * _TK, _TK)]
        mm = jnp.dot(xb, e.astype(jnp.bfloat16),
                     preferred_element_type=jnp.float32)
        ee = jnp.sum(e ** 2, axis=0, keepdims=True)
        dist = xx - 2.0 * mm + ee
        bmin = jnp.min(dist, axis=1, keepdims=True)
        io = lax.broadcasted_iota(jnp.int32, dist.shape, 1)
        cand = jnp.where(dist == bmin, io, kk)
        barg = jnp.min(cand, axis=1, keepdims=True) + k * _TK
        rmb = run_min.astype(jnp.bfloat16).astype(jnp.float32)
        keep = rmb <= bmin
        return (jnp.where(keep, rmb, bmin),
                jnp.where(keep, run_idx, barg))

    init = (jnp.full((_TN, 1), jnp.inf, jnp.float32),
            jnp.zeros((_TN, 1), jnp.int32))
    run_min, run_idx = lax.fori_loop(0, nk, body, init)
    idx_ref[...] = run_idx
    partial = jnp.sum(run_min, keepdims=True)
    acc = jnp.where(i == 0, partial, loss_ref[...] + partial)
    n_elem = nsteps * _TN * _DIM
    loss_ref[...] = jnp.where(i == nsteps - 1, acc * (1.0 / n_elem), acc)


def _dist_argmin(flat, embed):
    n = flat.shape[0]
    grid = n // _TN
    return pl.pallas_call(
        _dist_argmin_body,
        grid=(grid,),
        in_specs=[
            pl.BlockSpec((_TN, _DIM), lambda i: (i, 0)),
            pl.BlockSpec(embed.shape, lambda i: (0, 0)),
        ],
        out_specs=[
            pl.BlockSpec((_TN, 1), lambda i: (i, 0)),
            pl.BlockSpec((1, 1), lambda i: (0, 0)),
        ],
        out_shape=[
            jax.ShapeDtypeStruct((n, 1), jnp.int32),
            jax.ShapeDtypeStruct((1, 1), jnp.float32),
        ],
    )(flat, embed)


def _sc_gather(table, idx):
    n = idx.shape[0]
    d = table.shape[1]
    info = plsc.get_sparse_core_info()
    nw = info.num_cores * info.num_subcores
    b_per_w = n // nw
    mesh = plsc.VectorSubcoreMesh(core_axis_name="c", subcore_axis_name="s")

    @functools.partial(
        pl.kernel, mesh=mesh,
        compiler_params=pltpu.CompilerParams(use_tc_tiling_on_sc=False),
        out_type=jax.ShapeDtypeStruct((n, d), jnp.float32),
        scratch_types=[
            pltpu.VMEM((b_per_w,), jnp.int32),
            pltpu.VMEM((b_per_w, d), jnp.float32),
            pltpu.SemaphoreType.DMA,
        ],
    )
    def gather_kernel(table_hbm, idx_hbm, out_hbm, idx_v, rows_v, sem):
        wid = lax.axis_index("s") * info.num_cores + lax.axis_index("c")
        base = wid * b_per_w
        pltpu.sync_copy(idx_hbm.at[pl.ds(base, b_per_w)], idx_v)
        pltpu.async_copy(table_hbm.at[idx_v], rows_v, sem).wait()
        pltpu.sync_copy(rows_v, out_hbm.at[pl.ds(base, b_per_w)])

    return gather_kernel(table, idx)


def kernel(input, embed):
    b, c, h, w = input.shape
    flat = jnp.transpose(input, (0, 2, 3, 1)).reshape(-1, c)
    idx2d, loss2d = _dist_argmin(flat, embed)
    quant_flat = _sc_gather(embed.T, idx2d.reshape(-1))
    quantize = jnp.transpose(quant_flat.reshape(b, h, w, c), (0, 3, 1, 2))
    return (quantize, loss2d.reshape(()))


# speedup vs baseline: 1.2633x; 1.1339x over previous
import functools

import jax
import jax.numpy as jnp
from jax import lax
from jax.experimental import pallas as pl
from jax.experimental.pallas import tpu as pltpu
from jax.experimental.pallas import tpu_sc as plsc

_DIM = 32
_TN = 512
_TK = 2048


def _dist_argmin_body(x_ref, e_ref, idx_ref, loss_ref):
    i = pl.program_id(0)
    nsteps = pl.num_programs(0)
    x = x_ref[...]
    xx = jnp.sum(x ** 2, axis=1, keepdims=True)
    kk = e_ref.shape[1]
    nk = kk // _TK

    xb = x.astype(jnp.bfloat16)

    run_min = None
    run_idx = None
    for k in range(nk):
        e = e_ref[:, k * _TK:(k + 1) * _TK]
        mm2 = jnp.dot(xb, (-2.0 * e).astype(jnp.bfloat16),
                      preferred_element_type=jnp.float32)
        ee = jnp.sum(e ** 2, axis=0, keepdims=True)
        dist = xx + mm2 + ee
        bmin = jnp.min(dist, axis=1, keepdims=True)
        io = lax.broadcasted_iota(jnp.int32, dist.shape, 1)
        cand = jnp.where(dist == bmin, io, kk)
        barg = jnp.min(cand, axis=1, keepdims=True) + k * _TK
        if run_min is None:
            run_min, run_idx = bmin, barg
        else:
            rmb = run_min.astype(jnp.bfloat16).astype(jnp.float32)
            keep = rmb <= bmin
            run_min = jnp.where(keep, rmb, bmin)
            run_idx = jnp.where(keep, run_idx, barg)
    idx_ref[...] = run_idx
    partial = jnp.sum(run_min, keepdims=True)
    acc = jnp.where(i == 0, partial, loss_ref[...] + partial)
    n_elem = nsteps * _TN * _DIM
    loss_ref[...] = jnp.where(i == nsteps - 1, acc * (1.0 / n_elem), acc)


def _dist_argmin(flat, embed):
    n = flat.shape[0]
    grid = n // _TN
    return pl.pallas_call(
        _dist_argmin_body,
        grid=(grid,),
        in_specs=[
            pl.BlockSpec((_TN, _DIM), lambda i: (i, 0)),
            pl.BlockSpec(embed.shape, lambda i: (0, 0)),
        ],
        out_specs=[
            pl.BlockSpec((_TN, 1), lambda i: (i, 0)),
            pl.BlockSpec((1, 1), lambda i: (0, 0)),
        ],
        out_shape=[
            jax.ShapeDtypeStruct((n, 1), jnp.int32),
            jax.ShapeDtypeStruct((1, 1), jnp.float32),
        ],
    )(flat, embed)


def _sc_gather(table, idx):
    n = idx.shape[0]
    d = table.shape[1]
    info = plsc.get_sparse_core_info()
    nw = info.num_cores * info.num_subcores
    b_per_w = n // nw
    mesh = plsc.VectorSubcoreMesh(core_axis_name="c", subcore_axis_name="s")

    @functools.partial(
        pl.kernel, mesh=mesh,
        compiler_params=pltpu.CompilerParams(use_tc_tiling_on_sc=False),
        out_type=jax.ShapeDtypeStruct((n, d), jnp.float32),
        scratch_types=[
            pltpu.VMEM((b_per_w,), jnp.int32),
            pltpu.VMEM((b_per_w, d), jnp.float32),
            pltpu.SemaphoreType.DMA,
        ],
    )
    def gather_kernel(table_hbm, idx_hbm, out_hbm, idx_v, rows_v, sem):
        wid = lax.axis_index("s") * info.num_cores + lax.axis_index("c")
        base = wid * b_per_w
        pltpu.sync_copy(idx_hbm.at[pl.ds(base, b_per_w)], idx_v)
        pltpu.async_copy(table_hbm.at[idx_v], rows_v, sem).wait()
        pltpu.sync_copy(rows_v, out_hbm.at[pl.ds(base, b_per_w)])

    return gather_kernel(table, idx)


def kernel(input, embed):
    b, c, h, w = input.shape
    flat = jnp.transpose(input, (0, 2, 3, 1)).reshape(-1, c)
    idx2d, loss2d = _dist_argmin(flat, embed)
    quant_flat = _sc_gather(embed.T, idx2d.reshape(-1))
    quantize = jnp.transpose(quant_flat.reshape(b, h, w, c), (0, 3, 1, 2))
    return (quantize, loss2d.reshape(()))


# speedup vs baseline: 1.3552x; 1.0727x over previous
import functools

import jax
import jax.numpy as jnp
from jax import lax
from jax.experimental import pallas as pl
from jax.experimental.pallas import tpu as pltpu
from jax.experimental.pallas import tpu_sc as plsc

_DIM = 32
_TN = 512
_TK = 2048


def _dist_argmin_body(x_ref, e_ref, idx_ref, loss_ref):
    i = pl.program_id(0)
    nsteps = pl.num_programs(0)
    x = x_ref[...]
    xx = jnp.sum(x ** 2, axis=1, keepdims=True)
    kk = e_ref.shape[1]
    nk = kk // _TK

    xb = x.astype(jnp.bfloat16)

    run_min = None
    run_idx = None
    for k in range(nk):
        e = e_ref[:, k * _TK:(k + 1) * _TK]
        mm2 = jnp.dot(xb, (-2.0 * e).astype(jnp.bfloat16),
                      preferred_element_type=jnp.float32)
        ee = jnp.sum(e ** 2, axis=0, keepdims=True)
        dist = xx + mm2 + ee
        v = dist
        idxf = lax.broadcasted_iota(jnp.int32, (1, _TK), 1).astype(jnp.float32)
        width = _TK
        while width > 128:
            half = width // 2
            a, b2 = v[:, :half], v[:, half:]
            ia, ib = idxf[:, :half], idxf[:, half:]
            c = a <= b2
            v = jnp.minimum(a, b2)
            idxf = jnp.where(c, ia, ib)
            width = half
        bmin = jnp.min(v, axis=1, keepdims=True)
        cand = jnp.where(v == bmin, idxf, float(kk))
        bargf = jnp.min(cand, axis=1, keepdims=True)
        barg = bargf.astype(jnp.int32) + k * _TK
        if run_min is None:
            run_min, run_idx = bmin, barg
        else:
            rmb = run_min.astype(jnp.bfloat16).astype(jnp.float32)
            keep = rmb <= bmin
            run_min = jnp.where(keep, rmb, bmin)
            run_idx = jnp.where(keep, run_idx, barg)
    idx_ref[...] = run_idx
    partial = jnp.sum(run_min, keepdims=True)
    acc = jnp.where(i == 0, partial, loss_ref[...] + partial)
    n_elem = nsteps * _TN * _DIM
    loss_ref[...] = jnp.where(i == nsteps - 1, acc * (1.0 / n_elem), acc)


def _dist_argmin(flat, embed):
    n = flat.shape[0]
    grid = n // _TN
    return pl.pallas_call(
        _dist_argmin_body,
        grid=(grid,),
        in_specs=[
            pl.BlockSpec((_TN, _DIM), lambda i: (i, 0)),
            pl.BlockSpec(embed.shape, lambda i: (0, 0)),
        ],
        out_specs=[
            pl.BlockSpec((_TN, 1), lambda i: (i, 0)),
            pl.BlockSpec((1, 1), lambda i: (0, 0)),
        ],
        out_shape=[
            jax.ShapeDtypeStruct((n, 1), jnp.int32),
            jax.ShapeDtypeStruct((1, 1), jnp.float32),
        ],
    )(flat, embed)


def _sc_gather(table, idx):
    n = idx.shape[0]
    d = table.shape[1]
    info = plsc.get_sparse_core_info()
    nw = info.num_cores * info.num_subcores
    b_per_w = n // nw
    mesh = plsc.VectorSubcoreMesh(core_axis_name="c", subcore_axis_name="s")

    @functools.partial(
        pl.kernel, mesh=mesh,
        compiler_params=pltpu.CompilerParams(use_tc_tiling_on_sc=False),
        out_type=jax.ShapeDtypeStruct((n, d), jnp.float32),
        scratch_types=[
            pltpu.VMEM((b_per_w,), jnp.int32),
            pltpu.VMEM((b_per_w, d), jnp.float32),
            pltpu.SemaphoreType.DMA,
        ],
    )
    def gather_kernel(table_hbm, idx_hbm, out_hbm, idx_v, rows_v, sem):
        wid = lax.axis_index("s") * info.num_cores + lax.axis_index("c")
        base = wid * b_per_w
        pltpu.sync_copy(idx_hbm.at[pl.ds(base, b_per_w)], idx_v)
        pltpu.async_copy(table_hbm.at[idx_v], rows_v, sem).wait()
        pltpu.sync_copy(rows_v, out_hbm.at[pl.ds(base, b_per_w)])

    return gather_kernel(table, idx)


def kernel(input, embed):
    b, c, h, w = input.shape
    flat = jnp.transpose(input, (0, 2, 3, 1)).reshape(-1, c)
    idx2d, loss2d = _dist_argmin(flat, embed)
    quant_flat = _sc_gather(embed.T, idx2d.reshape(-1))
    quantize = jnp.transpose(quant_flat.reshape(b, h, w, c), (0, 3, 1, 2))
    return (quantize, loss2d.reshape(()))


# speedup vs baseline: 1.4227x; 1.0498x over previous
import functools

import jax
import jax.numpy as jnp
from jax import lax
from jax.experimental import pallas as pl
from jax.experimental.pallas import tpu as pltpu
from jax.experimental.pallas import tpu_sc as plsc

_DIM = 32
_TN = 1024
_TK = 2048


def _dist_argmin_body(x_ref, e_ref, idx_ref, loss_ref):
    i = pl.program_id(0)
    nsteps = pl.num_programs(0)
    x = x_ref[...]
    xx = jnp.sum(x ** 2, axis=1, keepdims=True)
    kk = e_ref.shape[1]
    nk = kk // _TK

    xb = x.astype(jnp.bfloat16)
    e_all = e_ref[...]
    eb_all = (-2.0 * e_all).astype(jnp.bfloat16)
    ee_all = jnp.sum(e_all ** 2, axis=0, keepdims=True)

    run_min = None
    run_idx = None
    for k in range(nk):
        mm2 = jnp.dot(xb, eb_all[:, k * _TK:(k + 1) * _TK],
                      preferred_element_type=jnp.float32)
        ee = ee_all[:, k * _TK:(k + 1) * _TK]
        dist = xx + mm2 + ee
        v = dist
        idxf = lax.broadcasted_iota(jnp.int32, (1, _TK), 1).astype(jnp.float32)
        width = _TK
        while width > 128:
            half = width // 2
            a, b2 = v[:, :half], v[:, half:]
            ia, ib = idxf[:, :half], idxf[:, half:]
            c = a <= b2
            v = jnp.minimum(a, b2)
            idxf = jnp.where(c, ia, ib)
            width = half
        bmin = jnp.min(v, axis=1, keepdims=True)
        cand = jnp.where(v == bmin, idxf, float(kk))
        bargf = jnp.min(cand, axis=1, keepdims=True)
        barg = bargf + float(k * _TK)
        if run_min is None:
            run_min, run_idx = bmin, barg
        else:
            rmb = run_min.astype(jnp.bfloat16).astype(jnp.float32)
            keep = rmb <= bmin
            run_min = jnp.where(keep, rmb, bmin)
            run_idx = jnp.where(keep, run_idx, barg)
    idx_ref[...] = run_idx.astype(jnp.int32)
    partial = jnp.sum(run_min, keepdims=True)
    acc = jnp.where(i == 0, partial, loss_ref[...] + partial)
    n_elem = nsteps * _TN * _DIM
    loss_ref[...] = jnp.where(i == nsteps - 1, acc * (1.0 / n_elem), acc)


def _dist_argmin(flat, embed):
    n = flat.shape[0]
    grid = n // _TN
    return pl.pallas_call(
        _dist_argmin_body,
        grid=(grid,),
        in_specs=[
            pl.BlockSpec((_TN, _DIM), lambda i: (i, 0)),
            pl.BlockSpec(embed.shape, lambda i: (0, 0)),
        ],
        out_specs=[
            pl.BlockSpec((_TN, 1), lambda i: (i, 0)),
            pl.BlockSpec((1, 1), lambda i: (0, 0)),
        ],
        out_shape=[
            jax.ShapeDtypeStruct((n, 1), jnp.int32),
            jax.ShapeDtypeStruct((1, 1), jnp.float32),
        ],
    )(flat, embed)


def _sc_gather(table, idx):
    n = idx.shape[0]
    d = table.shape[1]
    info = plsc.get_sparse_core_info()
    nw = info.num_cores * info.num_subcores
    b_per_w = n // nw
    mesh = plsc.VectorSubcoreMesh(core_axis_name="c", subcore_axis_name="s")

    @functools.partial(
        pl.kernel, mesh=mesh,
        compiler_params=pltpu.CompilerParams(use_tc_tiling_on_sc=False),
        out_type=jax.ShapeDtypeStruct((n, d), jnp.float32),
        scratch_types=[
            pltpu.VMEM((b_per_w,), jnp.int32),
            pltpu.VMEM((b_per_w, d), jnp.float32),
            pltpu.SemaphoreType.DMA,
        ],
    )
    def gather_kernel(table_hbm, idx_hbm, out_hbm, idx_v, rows_v, sem):
        wid = lax.axis_index("s") * info.num_cores + lax.axis_index("c")
        base = wid * b_per_w
        pltpu.sync_copy(idx_hbm.at[pl.ds(base, b_per_w)], idx_v)
        pltpu.async_copy(table_hbm.at[idx_v], rows_v, sem).wait()
        pltpu.sync_copy(rows_v, out_hbm.at[pl.ds(base, b_per_w)])

    return gather_kernel(table, idx)


def kernel(input, embed):
    b, c, h, w = input.shape
    flat = jnp.transpose(input, (0, 2, 3, 1)).reshape(-1, c)
    idx2d, loss2d = _dist_argmin(flat, embed)
    quant_flat = _sc_gather(embed.T, idx2d.reshape(-1))
    quantize = jnp.transpose(quant_flat.reshape(b, h, w, c), (0, 3, 1, 2))
    return (quantize, loss2d.reshape(()))
